# baseline (speedup 1.0000x reference)
import functools

import jax
import jax.numpy as jnp
from jax import lax
from jax.experimental import pallas as pl
from jax.experimental.pallas import tpu as pltpu
from jax.experimental.pallas import tpu_sc as plsc

_ALPHA = 4


@functools.lru_cache(maxsize=None)
def _make_sc_gather(C, T, H, W):
    S = T // _ALPHA
    info = plsc.get_sparse_core_info()
    NW = info.num_cores * info.num_subcores
    NFR = C * S
    CHN = 1
    while ((NFR * CHN) % NW != 0 or (H // CHN) * W > 49152
           or H % CHN != 0 or (H // CHN) % 8 != 0):
        CHN += 1
    ROWS = H // CHN
    PPW = (NFR * CHN) // NW

    mesh = plsc.VectorSubcoreMesh(core_axis_name="c", subcore_axis_name="s")

    @functools.partial(
        pl.kernel,
        mesh=mesh,
        out_type=jax.ShapeDtypeStruct((C, S, H, W), jnp.float32),
        scratch_types=[
            pltpu.VMEM((ROWS, W), jnp.float32),
            pltpu.VMEM((ROWS, W), jnp.float32),
            pltpu.SemaphoreType.DMA,
            pltpu.SemaphoreType.DMA,
            pltpu.SemaphoreType.DMA,
            pltpu.SemaphoreType.DMA,
        ],
    )
    def gather(frames_hbm, out_hbm, buf0, buf1, isem0, isem1, osem0, osem1):
        wid = lax.axis_index("s") * info.num_cores + lax.axis_index("c")
        bufs = (buf0, buf1)
        isems = (isem0, isem1)
        osems = (osem0, osem1)

        def coords(p):
            pid = wid * PPW + p
            c = pid // (S * CHN)
            rem = pid % (S * CHN)
            j = rem // CHN
            k = rem % CHN
            t = (j * (T - 1)) // (S - 1)
            return c, t, j, k

        in_cp = [None] * PPW
        out_cp = [None] * PPW
        for p in range(PPW):
            s = p % 2
            c, t, _, k = coords(p)
            if p >= 2:
                out_cp[p - 2].wait()
            in_cp[p] = pltpu.make_async_copy(
                frames_hbm.at[c, t, pl.ds(k * ROWS, ROWS), :],
                bufs[s], isems[s])
            in_cp[p].start()
            if p >= 1:
                c, _, j, k = coords(p - 1)
                in_cp[p - 1].wait()
                out_cp[p - 1] = pltpu.make_async_copy(
                    bufs[(p - 1) % 2],
                    out_hbm.at[c, j, pl.ds(k * ROWS, ROWS), :],
                    osems[(p - 1) % 2])
                out_cp[p - 1].start()
        c, _, j, k = coords(PPW - 1)
        in_cp[PPW - 1].wait()
        out_cp[PPW - 1] = pltpu.make_async_copy(
            bufs[(PPW - 1) % 2],
            out_hbm.at[c, j, pl.ds(k * ROWS, ROWS), :],
            osems[(PPW - 1) % 2])
        out_cp[PPW - 1].start()
        if PPW >= 2:
            out_cp[PPW - 2].wait()
        out_cp[PPW - 1].wait()

    return gather


@functools.lru_cache(maxsize=None)
def _make_tc_copy(C, T, H, W, BT=4, NB=8):
    NS = C * (T // BT)

    def slab(hbm, s):
        c, t = divmod(s, T // BT)
        return hbm.at[c, pl.ds(t * BT, BT)]

    def body(i_hbm, o_hbm, bufs, isems, osems):
        def cp_in(s, b):
            return pltpu.make_async_copy(slab(i_hbm, s), bufs.at[b],
                                         isems.at[b])

        def cp_out(s, b):
            return pltpu.make_async_copy(bufs.at[b], slab(o_hbm, s),
                                         osems.at[b])

        for s in range(min(NB, NS)):
            cp_in(s, s).start()
        for s in range(NS):
            b = s % NB
            cp_in(s, b).wait()
            cp_out(s, b).start()
            if s + NB < NS:
                cp_out(s, b).wait()
                cp_in(s + NB, b).start()
        for s in range(max(NS - NB, 0), NS):
            cp_out(s, s % NB).wait()

    return pl.pallas_call(
        body,
        in_specs=[pl.BlockSpec(memory_space=pltpu.MemorySpace.HBM)],
        out_specs=pl.BlockSpec(memory_space=pltpu.MemorySpace.HBM),
        out_shape=jax.ShapeDtypeStruct((C, T, H, W), jnp.float32),
        scratch_shapes=[
            pltpu.VMEM((NB, BT, H, W), jnp.float32),
            pltpu.SemaphoreType.DMA((NB,)),
            pltpu.SemaphoreType.DMA((NB,)),
        ],
    )


def kernel(frames):
    C, T, H, W = frames.shape
    slow = _make_sc_gather(C, T, H, W)(frames)
    fast = _make_tc_copy(C, T, H, W)(frames)
    return (slow, fast)


# speedup vs baseline: 1.0155x; 1.0155x over previous
import functools

import jax
import jax.numpy as jnp
from jax import lax
from jax.experimental import pallas as pl
from jax.experimental.pallas import tpu as pltpu
from jax.experimental.pallas import tpu_sc as plsc

_ALPHA = 4


@functools.lru_cache(maxsize=None)
def _make_sc_gather(C, T, H, W):
    S = T // _ALPHA
    info = plsc.get_sparse_core_info()
    NW = info.num_cores * info.num_subcores
    NFR = C * S
    CHN = 1
    while ((NFR * CHN) % NW != 0 or (H // CHN) * W > 49152
           or H % CHN != 0 or (H // CHN) % 8 != 0):
        CHN += 1
    ROWS = H // CHN
    PPW = (NFR * CHN) // NW

    mesh = plsc.VectorSubcoreMesh(core_axis_name="c", subcore_axis_name="s")

    @functools.partial(
        pl.kernel,
        mesh=mesh,
        out_type=jax.ShapeDtypeStruct((C, S, H, W), jnp.float32),
        scratch_types=[
            pltpu.VMEM((ROWS, W), jnp.float32),
            pltpu.VMEM((ROWS, W), jnp.float32),
            pltpu.SemaphoreType.DMA,
            pltpu.SemaphoreType.DMA,
            pltpu.SemaphoreType.DMA,
            pltpu.SemaphoreType.DMA,
        ],
    )
    def gather(frames_hbm, out_hbm, buf0, buf1, isem0, isem1, osem0, osem1):
        wid = lax.axis_index("s") * info.num_cores + lax.axis_index("c")
        bufs = (buf0, buf1)
        isems = (isem0, isem1)
        osems = (osem0, osem1)

        def coords(p):
            pid = wid * PPW + p
            c = pid // (S * CHN)
            rem = pid % (S * CHN)
            j = rem // CHN
            k = rem % CHN
            t = (j * (T - 1)) // (S - 1)
            return c, t, j, k

        in_cp = [None] * PPW
        out_cp = [None] * PPW
        for p in range(PPW):
            s = p % 2
            c, t, _, k = coords(p)
            if p >= 2:
                out_cp[p - 2].wait()
            in_cp[p] = pltpu.make_async_copy(
                frames_hbm.at[c, t, pl.ds(k * ROWS, ROWS), :],
                bufs[s], isems[s])
            in_cp[p].start()
            if p >= 1:
                c, _, j, k = coords(p - 1)
                in_cp[p - 1].wait()
                out_cp[p - 1] = pltpu.make_async_copy(
                    bufs[(p - 1) % 2],
                    out_hbm.at[c, j, pl.ds(k * ROWS, ROWS), :],
                    osems[(p - 1) % 2])
                out_cp[p - 1].start()
        c, _, j, k = coords(PPW - 1)
        in_cp[PPW - 1].wait()
        out_cp[PPW - 1] = pltpu.make_async_copy(
            bufs[(PPW - 1) % 2],
            out_hbm.at[c, j, pl.ds(k * ROWS, ROWS), :],
            osems[(PPW - 1) % 2])
        out_cp[PPW - 1].start()
        if PPW >= 2:
            out_cp[PPW - 2].wait()
        out_cp[PPW - 1].wait()

    return gather


@functools.lru_cache(maxsize=None)
def _make_tc_copy(C, T, H, W, BT=16):
    def body(i_ref, o_ref):
        o_ref[...] = i_ref[...]

    return pl.pallas_call(
        body,
        grid=(C, T // BT),
        in_specs=[pl.BlockSpec((1, BT, H, W), lambda c, t: (c, t, 0, 0))],
        out_specs=pl.BlockSpec((1, BT, H, W), lambda c, t: (c, t, 0, 0)),
        out_shape=jax.ShapeDtypeStruct((C, T, H, W), jnp.float32),
    )


def kernel(frames):
    C, T, H, W = frames.shape
    slow = _make_sc_gather(C, T, H, W)(frames)
    fast = _make_tc_copy(C, T, H, W)(frames)
    return (slow, fast)


# speedup vs baseline: 1.0157x; 1.0002x over previous
import functools

import jax
import jax.numpy as jnp
from jax import lax
from jax.experimental import pallas as pl
from jax.experimental.pallas import tpu as pltpu
from jax.experimental.pallas import tpu_sc as plsc

_ALPHA = 4


@functools.lru_cache(maxsize=None)
def _make_sc_gather(C, T, H, W):
    S = T // _ALPHA
    info = plsc.get_sparse_core_info()
    NW = info.num_cores * info.num_subcores
    NFR = C * S
    CHN = 1
    while ((NFR * CHN) % NW != 0 or (H // CHN) * W > 49152
           or H % CHN != 0 or (H // CHN) % 8 != 0):
        CHN += 1
    ROWS = H // CHN
    PPW = (NFR * CHN) // NW

    mesh = plsc.VectorSubcoreMesh(core_axis_name="c", subcore_axis_name="s")

    @functools.partial(
        pl.kernel,
        mesh=mesh,
        out_type=jax.ShapeDtypeStruct((C, S, H, W), jnp.float32),
        scratch_types=[
            pltpu.VMEM((ROWS, W), jnp.float32),
            pltpu.VMEM((ROWS, W), jnp.float32),
            pltpu.SemaphoreType.DMA,
            pltpu.SemaphoreType.DMA,
            pltpu.SemaphoreType.DMA,
            pltpu.SemaphoreType.DMA,
        ],
    )
    def gather(frames_hbm, out_hbm, buf0, buf1, isem0, isem1, osem0, osem1):
        wid = lax.axis_index("s") * info.num_cores + lax.axis_index("c")
        bufs = (buf0, buf1)
        isems = (isem0, isem1)
        osems = (osem0, osem1)

        def coords(p):
            pid = wid * PPW + p
            c = pid // (S * CHN)
            rem = pid % (S * CHN)
            j = rem // CHN
            k = rem % CHN
            t = (j * (T - 1)) // (S - 1)
            return c, t, j, k

        in_cp = [None] * PPW
        out_cp = [None] * PPW
        for p in range(PPW):
            s = p % 2
            c, t, _, k = coords(p)
            if p >= 2:
                out_cp[p - 2].wait()
            in_cp[p] = pltpu.make_async_copy(
                frames_hbm.at[c, t, pl.ds(k * ROWS, ROWS), :],
                bufs[s], isems[s])
            in_cp[p].start()
            if p >= 1:
                c, _, j, k = coords(p - 1)
                in_cp[p - 1].wait()
                out_cp[p - 1] = pltpu.make_async_copy(
                    bufs[(p - 1) % 2],
                    out_hbm.at[c, j, pl.ds(k * ROWS, ROWS), :],
                    osems[(p - 1) % 2])
                out_cp[p - 1].start()
        c, _, j, k = coords(PPW - 1)
        in_cp[PPW - 1].wait()
        out_cp[PPW - 1] = pltpu.make_async_copy(
            bufs[(PPW - 1) % 2],
            out_hbm.at[c, j, pl.ds(k * ROWS, ROWS), :],
            osems[(PPW - 1) % 2])
        out_cp[PPW - 1].start()
        if PPW >= 2:
            out_cp[PPW - 2].wait()
        out_cp[PPW - 1].wait()

    return gather


@functools.lru_cache(maxsize=None)
def _make_tc_copy(C, T, H, W, BT=16):
    def body(i_ref, o_ref):
        o_ref[...] = i_ref[...]

    return pl.pallas_call(
        body,
        grid=(C, T // BT),
        in_specs=[pl.BlockSpec((1, BT, H, W), lambda c, t: (c, t, 0, 0))],
        out_specs=pl.BlockSpec((1, BT, H, W), lambda c, t: (c, t, 0, 0)),
        out_shape=jax.ShapeDtypeStruct((C, T, H, W), jnp.float32),
    )


def kernel(frames):
    C, T, H, W = frames.shape
    fast = _make_tc_copy(C, T, H, W)(frames)
    slow = _make_sc_gather(C, T, H, W)(frames)
    return (slow, fast)


# speedup vs baseline: 1.0189x; 1.0031x over previous
import functools

import jax
import jax.numpy as jnp
from jax import lax
from jax.experimental import pallas as pl
from jax.experimental.pallas import tpu as pltpu
from jax.experimental.pallas import tpu_sc as plsc

_ALPHA = 4


@functools.lru_cache(maxsize=None)
def _make_sc_gather(C, T, H, W):
    S = T // _ALPHA
    info = plsc.get_sparse_core_info()
    NW = info.num_cores * info.num_subcores
    NFR = C * S
    CHN = 1
    while ((NFR * CHN) % NW != 0 or (H // CHN) * W > 49152
           or H % CHN != 0 or (H // CHN) % 8 != 0):
        CHN += 1
    ROWS = H // CHN
    PPW = (NFR * CHN) // NW

    mesh = plsc.VectorSubcoreMesh(core_axis_name="c", subcore_axis_name="s")

    @functools.partial(
        pl.kernel,
        mesh=mesh,
        out_type=jax.ShapeDtypeStruct((C, S, H, W), jnp.float32),
        scratch_types=[
            pltpu.VMEM((ROWS, W), jnp.float32),
        ],
    )
    def gather(frames_hbm, out_hbm, buf):
        wid = lax.axis_index("s") * info.num_cores + lax.axis_index("c")

        def step(p, _):
            pid = wid * PPW + p
            c = pid // (S * CHN)
            rem = pid % (S * CHN)
            j = rem // CHN
            k = rem % CHN
            t = (j * (T - 1)) // (S - 1)
            pltpu.sync_copy(frames_hbm.at[c, t, pl.ds(k * ROWS, ROWS), :],
                            buf)
            pltpu.sync_copy(buf,
                            out_hbm.at[c, j, pl.ds(k * ROWS, ROWS), :])
            return _

        lax.fori_loop(0, PPW, step, 0)

    return gather


@functools.lru_cache(maxsize=None)
def _make_tc_copy(C, T, H, W, BT=16):
    def body(i_ref, o_ref):
        o_ref[...] = i_ref[...]

    return pl.pallas_call(
        body,
        grid=(C, T // BT),
        in_specs=[pl.BlockSpec((1, BT, H, W), lambda c, t: (c, t, 0, 0))],
        out_specs=pl.BlockSpec((1, BT, H, W), lambda c, t: (c, t, 0, 0)),
        out_shape=jax.ShapeDtypeStruct((C, T, H, W), jnp.float32),
    )


def kernel(frames):
    C, T, H, W = frames.shape
    fast = _make_tc_copy(C, T, H, W)(frames)
    slow = _make_sc_gather(C, T, H, W)(frames)
    return (slow, fast)


# speedup vs baseline: 1.0338x; 1.0147x over previous
import functools

import jax
import jax.numpy as jnp
from jax import lax
from jax.experimental import pallas as pl
from jax.experimental.pallas import tpu as pltpu
from jax.experimental.pallas import tpu_sc as plsc

_ALPHA = 4


@functools.lru_cache(maxsize=None)
def _make_sc_gather(C, T, H, W):
    S = T // _ALPHA
    info = plsc.get_sparse_core_info()
    NW = info.num_cores * info.num_subcores
    NFR = C * S
    CHN = 1
    while ((NFR * CHN) % NW != 0 or (H // CHN) * W > 49152
           or H % CHN != 0 or (H // CHN) % 8 != 0):
        CHN += 1
    ROWS = H // CHN
    PPW = (NFR * CHN) // NW

    mesh = plsc.ScalarSubcoreMesh(axis_name="c", num_cores=info.num_cores)
    FPC = NFR // info.num_cores

    @functools.partial(
        pl.kernel,
        mesh=mesh,
        out_type=jax.ShapeDtypeStruct((C, S, H, W), jnp.float32),
        scratch_types=[
            pltpu.VMEM_SHARED((2, H, W), jnp.float32),
            pltpu.SemaphoreType.DMA,
            pltpu.SemaphoreType.DMA,
            pltpu.SemaphoreType.DMA,
            pltpu.SemaphoreType.DMA,
        ],
    )
    def gather(frames_hbm, out_hbm, buf, isem0, isem1, osem0, osem1):
        cid = lax.axis_index("c")
        isems = (isem0, isem1)
        osems = (osem0, osem1)

        def coords(p):
            pid = cid * FPC + p
            c = pid // S
            j = pid % S
            t = (j * (T - 1)) // (S - 1)
            return c, t, j

        in_cp = [None] * FPC
        out_cp = [None] * FPC
        for p in range(FPC):
            b = p % 2
            c, t, _ = coords(p)
            if p >= 2:
                out_cp[p - 2].wait()
            in_cp[p] = pltpu.make_async_copy(
                frames_hbm.at[c, t], buf.at[b], isems[b])
            in_cp[p].start()
            if p >= 1:
                c, _, j = coords(p - 1)
                in_cp[p - 1].wait()
                out_cp[p - 1] = pltpu.make_async_copy(
                    buf.at[(p - 1) % 2], out_hbm.at[c, j],
                    osems[(p - 1) % 2])
                out_cp[p - 1].start()
        c, _, j = coords(FPC - 1)
        in_cp[FPC - 1].wait()
        out_cp[FPC - 1] = pltpu.make_async_copy(
            buf.at[(FPC - 1) % 2], out_hbm.at[c, j], osems[(FPC - 1) % 2])
        out_cp[FPC - 1].start()
        out_cp[FPC - 2].wait()
        out_cp[FPC - 1].wait()

    return gather


@functools.lru_cache(maxsize=None)
def _make_tc_copy(C, T, H, W, BT=16):
    def body(i_ref, o_ref):
        o_ref[...] = i_ref[...]

    return pl.pallas_call(
        body,
        grid=(C, T // BT),
        in_specs=[pl.BlockSpec((1, BT, H, W), lambda c, t: (c, t, 0, 0))],
        out_specs=pl.BlockSpec((1, BT, H, W), lambda c, t: (c, t, 0, 0)),
        out_shape=jax.ShapeDtypeStruct((C, T, H, W), jnp.float32),
    )


def kernel(frames):
    C, T, H, W = frames.shape
    fast = _make_tc_copy(C, T, H, W)(frames)
    slow = _make_sc_gather(C, T, H, W)(frames)
    return (slow, fast)
